# Initial kernel scaffold; baseline (speedup 1.0000x reference)
import functools

import jax
import jax.numpy as jnp
from jax import lax
from jax.experimental import pallas as pl
from jax.experimental.pallas import tpu as pltpu
from jax.experimental.pallas import tpu_sc as plsc

NC = 2
NS = 16
NW = NC * NS


def _prep_body(x_ref, wf_ref, asrc_ref, adst_ref, h_ref, t1_ref, t2_ref, m_ref):
    h = jnp.dot(x_ref[...], wf_ref[...], preferred_element_type=jnp.float32)
    h_ref[...] = h
    t1 = jnp.dot(h, asrc_ref[...], preferred_element_type=jnp.float32)
    t2 = jnp.dot(h, adst_ref[...], preferred_element_type=jnp.float32)
    t1_ref[...] = t1
    t2_ref[...] = t2
    s = jnp.max(t1, axis=0, keepdims=True) + jnp.max(t2, axis=0, keepdims=True)
    m = jnp.maximum(s, 0.2 * s)
    m_ref[...] = jnp.broadcast_to(m, (8, 16))


def _fin_body(acc_ref, den_ref, sel_ref, bias_ref, out_ref):
    den8 = den_ref[0][:, :8] + den_ref[1][:, :8]
    rep = jnp.dot(den8, sel_ref[...], preferred_element_type=jnp.float32)
    out_ref[...] = (acc_ref[0] + acc_ref[1]) / (rep + 1e-16) + bias_ref[...]


def _sc_edge_body(n_nodes, d_model, ept, g_size,
                  src_hbm, dst_hbm, t1_hbm, t2_hbm, h_hbm, m_hbm,
                  acc_hbm, den_hbm,
                  out_sh, den_sh,
                  sidx, didx, t1b, t2b, hb, zb, msgb, mb, zden,
                  sem0, sem1, sem2, sem3, sem4):
    c = lax.axis_index("c")
    s = lax.axis_index("s")
    wid = c * NS + s
    ebase = wid * ept
    n_groups = ept // g_size
    rpt = n_nodes // NS
    rbase = s * rpt
    n_seg = d_model // 16

    zvec = jnp.zeros((16,), jnp.float32)

    def zero_den_row(i, _):
        zden[i] = zvec
        return 0
    lax.fori_loop(0, rpt, zero_den_row, 0)
    pltpu.sync_copy(zden, den_sh.at[pl.ds(rbase, rpt)])

    def zero_msg_row(i, _):
        for j in range(n_seg):
            msgb[i, pl.ds(j * 16, 16)] = zvec
        return 0
    lax.fori_loop(0, g_size, zero_msg_row, 0)
    n_zcp = rpt // g_size
    for k in range(n_zcp):
        pltpu.sync_copy(msgb, out_sh.at[pl.ds(rbase + k * g_size, g_size)])
    rem = rpt - n_zcp * g_size
    if rem > 0:
        pltpu.sync_copy(msgb.at[pl.ds(0, rem)],
                        out_sh.at[pl.ds(rbase + n_zcp * g_size, rem)])
    plsc.subcore_barrier()

    pltpu.sync_copy(m_hbm.at[0], mb)
    mvec = mb[...]

    def group(g, _):
        base = ebase + g * g_size
        pltpu.sync_copy(src_hbm.at[pl.ds(base, g_size)], sidx)
        pltpu.sync_copy(dst_hbm.at[pl.ds(base, g_size)], didx)
        cp1 = pltpu.async_copy(t1_hbm.at[sidx], t1b, sem0)
        cp2 = pltpu.async_copy(t2_hbm.at[didx], t2b, sem1)
        cp3 = pltpu.async_copy(h_hbm.at[sidx], hb, sem2)
        cp1.wait()
        cp2.wait()
        cp3.wait()

        def edge(i, _):
            sv = t1b[i] + t2b[i]
            z = jnp.exp(jnp.maximum(sv, 0.2 * sv) - mvec)
            zb[i] = z
            for j in range(n_seg):
                scale = zb[i, j]
                msgb[i, pl.ds(j * 16, 16)] = hb[i, pl.ds(j * 16, 16)] * scale
            return 0
        lax.fori_loop(0, g_size, edge, 0)

        cp4 = pltpu.async_copy(zb, den_sh.at[didx], sem3, add=True)
        cp5 = pltpu.async_copy(msgb, out_sh.at[didx], sem4, add=True)
        cp4.wait()
        cp5.wait()
        return 0
    lax.fori_loop(0, n_groups, group, 0)

    plsc.subcore_barrier()
    pltpu.sync_copy(out_sh.at[pl.ds(rbase, rpt)],
                    acc_hbm.at[c, pl.ds(rbase, rpt)])
    pltpu.sync_copy(den_sh.at[pl.ds(rbase, rpt)],
                    den_hbm.at[c, pl.ds(rbase, rpt)])


def kernel(x, edge_index, W, a_src, a_dst, bias):
    n, f_in = x.shape
    h_heads, _, f_out = W.shape
    e_edges = edge_index.shape[1]
    d = h_heads * f_out

    wf = W.transpose(1, 0, 2).reshape(f_in, d)
    rows = jnp.arange(d)
    hcol = rows // f_out
    a_src_m = jnp.zeros((d, 16), jnp.float32).at[rows, hcol].set(a_src.reshape(-1))
    a_dst_m = jnp.zeros((d, 16), jnp.float32).at[rows, hcol].set(a_dst.reshape(-1))
    src = edge_index[0].astype(jnp.int32)
    dst = edge_index[1].astype(jnp.int32)
    sel = jnp.repeat(jnp.eye(h_heads, dtype=jnp.float32), f_out, axis=1)
    bias2 = bias.reshape(1, d).astype(jnp.float32)

    h_arr, t1, t2, m = pl.pallas_call(
        _prep_body,
        out_shape=[
            jax.ShapeDtypeStruct((n, d), jnp.float32),
            jax.ShapeDtypeStruct((n, 16), jnp.float32),
            jax.ShapeDtypeStruct((n, 16), jnp.float32),
            jax.ShapeDtypeStruct((8, 16), jnp.float32),
        ],
    )(x, wf, a_src_m, a_dst_m)

    ept = e_edges // NW
    g_size = 80
    assert ept % g_size == 0 and n % NS == 0

    mesh = plsc.VectorSubcoreMesh(core_axis_name="c", subcore_axis_name="s")
    sc_fn = pl.kernel(
        functools.partial(_sc_edge_body, n, d, ept, g_size),
        out_type=[
            jax.ShapeDtypeStruct((NC, n, d), jnp.float32),
            jax.ShapeDtypeStruct((NC, n, 16), jnp.float32),
        ],
        mesh=mesh,
        scratch_types=[
            pltpu.VMEM_SHARED((n, d), jnp.float32),
            pltpu.VMEM_SHARED((n, 16), jnp.float32),
            pltpu.VMEM((g_size,), jnp.int32),
            pltpu.VMEM((g_size,), jnp.int32),
            pltpu.VMEM((g_size, 16), jnp.float32),
            pltpu.VMEM((g_size, 16), jnp.float32),
            pltpu.VMEM((g_size, d), jnp.float32),
            pltpu.VMEM((g_size, 16), jnp.float32),
            pltpu.VMEM((g_size, d), jnp.float32),
            pltpu.VMEM((16,), jnp.float32),
            pltpu.VMEM((n // NS, 16), jnp.float32),
            pltpu.SemaphoreType.DMA,
            pltpu.SemaphoreType.DMA,
            pltpu.SemaphoreType.DMA,
            pltpu.SemaphoreType.DMA,
            pltpu.SemaphoreType.DMA,
        ],
    )
    acc, den = sc_fn(src, dst, t1, t2, h_arr, m)

    out = pl.pallas_call(
        _fin_body,
        out_shape=jax.ShapeDtypeStruct((n, d), jnp.float32),
    )(acc, den, sel, bias2)
    return out


# speedup vs baseline: 44.7632x; 44.7632x over previous
import functools

import jax
import jax.numpy as jnp
from jax import lax
from jax.experimental import pallas as pl
from jax.experimental.pallas import tpu as pltpu
from jax.experimental.pallas import tpu_sc as plsc

NC = 2
NS = 16
NW = NC * NS


def _prep_body(x_ref, wf_ref, asrc_ref, adst_ref, h_ref, t1_ref, t2_ref, m_ref):
    h = jnp.dot(x_ref[...], wf_ref[...], preferred_element_type=jnp.float32)
    h_ref[...] = h
    t1 = jnp.dot(h, asrc_ref[...], preferred_element_type=jnp.float32)
    t2 = jnp.dot(h, adst_ref[...], preferred_element_type=jnp.float32)
    t1_ref[...] = t1
    t2_ref[...] = t2
    s = jnp.max(t1, axis=0, keepdims=True) + jnp.max(t2, axis=0, keepdims=True)
    m = jnp.maximum(s, 0.2 * s)
    m_ref[...] = jnp.broadcast_to(m, (8, 16))


def _fin_body(acc_ref, den_ref, sel_ref, bias_ref, out_ref):
    den8 = den_ref[0][:, :8] + den_ref[1][:, :8]
    rep = jnp.dot(den8, sel_ref[...], preferred_element_type=jnp.float32)
    out_ref[...] = (acc_ref[0] + acc_ref[1]) / (rep + 1e-16) + bias_ref[...]


def _sc_edge_body(n_nodes, d_model, ept, g_size,
                  src_hbm, dst_hbm, t1_hbm, t2_hbm, h_hbm, m_hbm,
                  acc_hbm, den_hbm,
                  out_sh, den_sh,
                  sidx, didx, t1b, t2b, hb, zb, msgb, mb, zden,
                  sem0, sem1, sem2, sem3, sem4):
    c = lax.axis_index("c")
    s = lax.axis_index("s")
    wid = c * NS + s
    ebase = wid * ept
    n_groups = ept // g_size
    n_seg = d_model // 16
    rpt = 8 * (n_nodes // (8 * NS))
    rbase = s * rpt
    tail = n_nodes - NS * rpt
    tail_base = NS * rpt

    zvec = jnp.zeros((16,), jnp.float32)

    def zero_den_row(i, _):
        zden[i] = zvec
        return 0
    lax.fori_loop(0, rpt, zero_den_row, 0)
    pltpu.sync_copy(zden, den_sh.at[pl.ds(rbase, rpt)])

    def zero_msg_row(i, _):
        for j in range(n_seg):
            msgb[i, pl.ds(j * 16, 16)] = zvec
        return 0
    lax.fori_loop(0, g_size, zero_msg_row, 0)
    n_zcp = rpt // g_size
    for k in range(n_zcp):
        pltpu.sync_copy(msgb, out_sh.at[pl.ds(rbase + k * g_size, g_size)])
    rem = rpt - n_zcp * g_size
    if rem > 0:
        pltpu.sync_copy(msgb.at[pl.ds(0, rem)],
                        out_sh.at[pl.ds(rbase + n_zcp * g_size, rem)])
    if tail > 0:
        @pl.when(s == 0)
        def _zero_tail():
            pltpu.sync_copy(zden.at[pl.ds(0, tail)],
                            den_sh.at[pl.ds(tail_base, tail)])
            pltpu.sync_copy(msgb.at[pl.ds(0, tail)],
                            out_sh.at[pl.ds(tail_base, tail)])
    plsc.subcore_barrier()

    pltpu.sync_copy(m_hbm.at[0], mb)
    mvec = mb[...]

    def group(g, _):
        base = ebase + g * g_size
        pltpu.sync_copy(src_hbm.at[pl.ds(base, g_size)], sidx)
        pltpu.sync_copy(dst_hbm.at[pl.ds(base, g_size)], didx)
        cp1 = pltpu.async_copy(t1_hbm.at[sidx], t1b, sem0)
        cp2 = pltpu.async_copy(t2_hbm.at[didx], t2b, sem1)
        cp3 = pltpu.async_copy(h_hbm.at[sidx], hb, sem2)
        cp1.wait()
        cp2.wait()
        cp3.wait()

        def edge(i, _):
            sv = t1b[i] + t2b[i]
            z = jnp.exp(jnp.maximum(sv, 0.2 * sv) - mvec)
            zb[i] = z
            for j in range(n_seg):
                scale = z[j]
                msgb[i, pl.ds(j * 16, 16)] = hb[i, pl.ds(j * 16, 16)] * scale
            return 0
        lax.fori_loop(0, g_size, edge, 0)

        cp4 = pltpu.async_copy(zb, den_sh.at[didx], sem3, add=True)
        cp5 = pltpu.async_copy(msgb, out_sh.at[didx], sem4, add=True)
        cp4.wait()
        cp5.wait()
        return 0
    lax.fori_loop(0, n_groups, group, 0)

    plsc.subcore_barrier()
    pltpu.sync_copy(out_sh.at[pl.ds(rbase, rpt)],
                    acc_hbm.at[c, pl.ds(rbase, rpt)])
    pltpu.sync_copy(den_sh.at[pl.ds(rbase, rpt)],
                    den_hbm.at[c, pl.ds(rbase, rpt)])
    if tail > 0:
        @pl.when(s == 0)
        def _read_tail():
            pltpu.sync_copy(out_sh.at[pl.ds(tail_base, tail)],
                            acc_hbm.at[c, pl.ds(tail_base, tail)])
            pltpu.sync_copy(den_sh.at[pl.ds(tail_base, tail)],
                            den_hbm.at[c, pl.ds(tail_base, tail)])


def kernel(x, edge_index, W, a_src, a_dst, bias):
    n, f_in = x.shape
    h_heads, _, f_out = W.shape
    e_edges = edge_index.shape[1]
    d = h_heads * f_out

    wf = W.transpose(1, 0, 2).reshape(f_in, d)
    rows = jnp.arange(d)
    hcol = rows // f_out
    a_src_m = jnp.zeros((d, 16), jnp.float32).at[rows, hcol].set(a_src.reshape(-1))
    a_dst_m = jnp.zeros((d, 16), jnp.float32).at[rows, hcol].set(a_dst.reshape(-1))
    src = edge_index[0].astype(jnp.int32)
    dst = edge_index[1].astype(jnp.int32)
    sel = jnp.repeat(jnp.eye(h_heads, dtype=jnp.float32), f_out, axis=1)
    bias2 = bias.reshape(1, d).astype(jnp.float32)

    h_arr, t1, t2, m = pl.pallas_call(
        _prep_body,
        out_shape=[
            jax.ShapeDtypeStruct((n, d), jnp.float32),
            jax.ShapeDtypeStruct((n, 16), jnp.float32),
            jax.ShapeDtypeStruct((n, 16), jnp.float32),
            jax.ShapeDtypeStruct((8, 16), jnp.float32),
        ],
    )(x, wf, a_src_m, a_dst_m)

    ept = e_edges // NW
    g_size = 80
    assert ept % g_size == 0 and n % NS == 0

    mesh = plsc.VectorSubcoreMesh(core_axis_name="c", subcore_axis_name="s")
    sc_fn = pl.kernel(
        functools.partial(_sc_edge_body, n, d, ept, g_size),
        out_type=[
            jax.ShapeDtypeStruct((NC, n, d), jnp.float32),
            jax.ShapeDtypeStruct((NC, n, 16), jnp.float32),
        ],
        mesh=mesh,
        compiler_params=pltpu.CompilerParams(use_tc_tiling_on_sc=False),
        scratch_types=[
            pltpu.VMEM_SHARED((n, d), jnp.float32),
            pltpu.VMEM_SHARED((n, 16), jnp.float32),
            pltpu.VMEM((g_size,), jnp.int32),
            pltpu.VMEM((g_size,), jnp.int32),
            pltpu.VMEM((g_size, 16), jnp.float32),
            pltpu.VMEM((g_size, 16), jnp.float32),
            pltpu.VMEM((g_size, d), jnp.float32),
            pltpu.VMEM((g_size, 16), jnp.float32),
            pltpu.VMEM((g_size, d), jnp.float32),
            pltpu.VMEM((16,), jnp.float32),
            pltpu.VMEM((8 * (n // (8 * NS)), 16), jnp.float32),
            pltpu.SemaphoreType.DMA,
            pltpu.SemaphoreType.DMA,
            pltpu.SemaphoreType.DMA,
            pltpu.SemaphoreType.DMA,
            pltpu.SemaphoreType.DMA,
        ],
    )
    acc, den = sc_fn(src, dst, t1, t2, h_arr, m)

    out = pl.pallas_call(
        _fin_body,
        out_shape=jax.ShapeDtypeStruct((n, d), jnp.float32),
    )(acc, den, sel, bias2)
    return out


# speedup vs baseline: 65.7727x; 1.4693x over previous
import functools

import jax
import jax.numpy as jnp
from jax import lax
from jax.experimental import pallas as pl
from jax.experimental.pallas import tpu as pltpu
from jax.experimental.pallas import tpu_sc as plsc

NC = 2
NS = 16
NW = NC * NS


def _prep_body(x_ref, wf_ref, asrc_ref, adst_ref, h_ref, t1_ref, t2_ref, m_ref):
    h = jnp.dot(x_ref[...], wf_ref[...], preferred_element_type=jnp.float32)
    h_ref[...] = h
    t1 = jnp.dot(h, asrc_ref[...], preferred_element_type=jnp.float32)
    t2 = jnp.dot(h, adst_ref[...], preferred_element_type=jnp.float32)
    t1_ref[...] = t1
    t2_ref[...] = t2
    s = jnp.max(t1, axis=0, keepdims=True) + jnp.max(t2, axis=0, keepdims=True)
    m = jnp.maximum(s, 0.2 * s)
    m_ref[...] = jnp.broadcast_to(m, (8, 16))


def _fin_body(acc_ref, den_ref, sel_ref, bias_ref, out_ref):
    den8 = den_ref[0][:, :8] + den_ref[1][:, :8]
    rep = jnp.dot(den8, sel_ref[...], preferred_element_type=jnp.float32)
    out_ref[...] = (acc_ref[0] + acc_ref[1]) / (rep + 1e-16) + bias_ref[...]


def _sc_edge_body(n_nodes, d_model, ept, g_size,
                  src_hbm, dst_hbm, t1_hbm, t2_hbm, h_hbm, m_hbm,
                  acc_hbm, den_hbm,
                  out_sh, den_sh,
                  sidx_all, didx_all,
                  t1b0, t1b1, t2b0, t2b1,
                  hb0, hb1, zb0, zb1, mg0, mg1,
                  mb,
                  gs0, gs1, ss0, ss1):
    t1bs = (t1b0, t1b1)
    t2bs = (t2b0, t2b1)
    hbs = (hb0, hb1)
    zbs = (zb0, zb1)
    msgbs = (mg0, mg1)
    gsems = (gs0, gs1)
    ssems = (ss0, ss1)
    c = lax.axis_index("c")
    s = lax.axis_index("s")
    wid = c * NS + s
    n_groups = ept // g_size
    n_seg = d_model // 16
    msgb = mg0
    rpt = 8 * (n_nodes // (8 * NS))
    rbase = s * rpt
    tail = n_nodes - NS * rpt
    tail_base = NS * rpt

    zvec = jnp.zeros((16,), jnp.float32)

    def zero_z_row(i, _):
        zb0[i] = zvec
        return 0
    lax.fori_loop(0, g_size, zero_z_row, 0)

    def zero_msg_row(i, _):
        for j in range(n_seg):
            msgb[i, pl.ds(j * 16, 16)] = zvec
        return 0
    lax.fori_loop(0, g_size, zero_msg_row, 0)
    n_zcp = rpt // g_size
    for k in range(n_zcp):
        pltpu.sync_copy(zb0, den_sh.at[pl.ds(rbase + k * g_size, g_size)])
        pltpu.sync_copy(msgb, out_sh.at[pl.ds(rbase + k * g_size, g_size)])
    rem = rpt - n_zcp * g_size
    if rem > 0:
        pltpu.sync_copy(zb0.at[pl.ds(0, rem)],
                        den_sh.at[pl.ds(rbase + n_zcp * g_size, rem)])
        pltpu.sync_copy(msgb.at[pl.ds(0, rem)],
                        out_sh.at[pl.ds(rbase + n_zcp * g_size, rem)])
    if tail > 0:
        @pl.when(s == 0)
        def _zero_tail():
            pltpu.sync_copy(zb0.at[pl.ds(0, tail)],
                            den_sh.at[pl.ds(tail_base, tail)])
            pltpu.sync_copy(msgb.at[pl.ds(0, tail)],
                            out_sh.at[pl.ds(tail_base, tail)])
    plsc.subcore_barrier()

    pltpu.sync_copy(m_hbm.at[0], mb)
    mvec = mb[...]

    def fire_gathers(g, b):
        pltpu.async_copy(t1_hbm.at[sidx_all.at[g]], t1bs[b], gsems[b])
        pltpu.async_copy(t2_hbm.at[didx_all.at[g]], t2bs[b], gsems[b])
        pltpu.async_copy(h_hbm.at[sidx_all.at[g]], hbs[b], gsems[b])

    def wait_gathers(g, b):
        pltpu.make_async_copy(t1_hbm.at[sidx_all.at[g]], t1bs[b], gsems[b]).wait()
        pltpu.make_async_copy(t2_hbm.at[didx_all.at[g]], t2bs[b], gsems[b]).wait()
        pltpu.make_async_copy(h_hbm.at[sidx_all.at[g]], hbs[b], gsems[b]).wait()

    def fire_scatters(g, b):
        pltpu.async_copy(zbs[b], den_sh.at[didx_all.at[g]], ssems[b], add=True)
        pltpu.async_copy(msgbs[b], out_sh.at[didx_all.at[g]], ssems[b], add=True)

    def wait_scatters(g, b):
        pltpu.make_async_copy(zbs[b], den_sh.at[didx_all.at[g]], ssems[b]).wait()
        pltpu.make_async_copy(msgbs[b], out_sh.at[didx_all.at[g]], ssems[b]).wait()

    def compute(g, b):
        t1r, t2r, hr, zr, mr = t1bs[b], t2bs[b], hbs[b], zbs[b], msgbs[b]

        def edge(i, _):
            sv = t1r[i] + t2r[i]
            z = jnp.exp(jnp.maximum(sv, 0.2 * sv) - mvec)
            zr[i] = z
            for j in range(n_seg):
                scale = z[j]
                mr[i, pl.ds(j * 16, 16)] = hr[i, pl.ds(j * 16, 16)] * scale
            return 0
        lax.fori_loop(0, g_size, edge, 0)

    def step(g, b):
        wait_gathers(g, b)

        @pl.when(g >= 2)
        def _():
            wait_scatters(g - 2, b)
        compute(g, b)
        fire_scatters(g, b)

    n_phases = 2
    assert n_groups % n_phases == 0
    pg = n_groups // n_phases
    assert pg > 2
    for ph in range(n_phases):
        pltpu.sync_copy(src_hbm.at[wid, pl.ds(ph * pg, pg)], sidx_all)
        pltpu.sync_copy(dst_hbm.at[wid, pl.ds(ph * pg, pg)], didx_all)
        for b in range(2):
            fire_gathers(b, b)

        def body(i, _):
            for b in range(2):
                g = 2 * i + b
                step(g, b)

                @pl.when(g < pg - 2)
                def _():
                    fire_gathers(g + 2, b)
            return 0
        lax.fori_loop(0, pg // 2, body, 0)
        if pg % 2 == 1:
            step(pg - 1, (pg - 1) % 2)
        for g in (pg - 2, pg - 1):
            wait_scatters(g, g % 2)

    plsc.subcore_barrier()
    pltpu.sync_copy(out_sh.at[pl.ds(rbase, rpt)],
                    acc_hbm.at[c, pl.ds(rbase, rpt)])
    pltpu.sync_copy(den_sh.at[pl.ds(rbase, rpt)],
                    den_hbm.at[c, pl.ds(rbase, rpt)])
    if tail > 0:
        @pl.when(s == 0)
        def _read_tail():
            pltpu.sync_copy(out_sh.at[pl.ds(tail_base, tail)],
                            acc_hbm.at[c, pl.ds(tail_base, tail)])
            pltpu.sync_copy(den_sh.at[pl.ds(tail_base, tail)],
                            den_hbm.at[c, pl.ds(tail_base, tail)])


def kernel(x, edge_index, W, a_src, a_dst, bias):
    n, f_in = x.shape
    h_heads, _, f_out = W.shape
    e_edges = edge_index.shape[1]
    d = h_heads * f_out

    wf = W.transpose(1, 0, 2).reshape(f_in, d)
    rows = jnp.arange(d)
    hcol = rows // f_out
    a_src_m = jnp.zeros((d, 16), jnp.float32).at[rows, hcol].set(a_src.reshape(-1))
    a_dst_m = jnp.zeros((d, 16), jnp.float32).at[rows, hcol].set(a_dst.reshape(-1))
    src = edge_index[0].astype(jnp.int32)
    dst = edge_index[1].astype(jnp.int32)
    sel = jnp.repeat(jnp.eye(h_heads, dtype=jnp.float32), f_out, axis=1)
    bias2 = bias.reshape(1, d).astype(jnp.float32)

    h_arr, t1, t2, m = pl.pallas_call(
        _prep_body,
        out_shape=[
            jax.ShapeDtypeStruct((n, d), jnp.float32),
            jax.ShapeDtypeStruct((n, 16), jnp.float32),
            jax.ShapeDtypeStruct((n, 16), jnp.float32),
            jax.ShapeDtypeStruct((8, 16), jnp.float32),
        ],
    )(x, wf, a_src_m, a_dst_m)

    ept = e_edges // NW
    g_size = 40
    assert ept % g_size == 0 and n % NS == 0

    n_groups = ept // g_size
    src3 = src.reshape(NW, n_groups, g_size)
    dst3 = dst.reshape(NW, n_groups, g_size)

    mesh = plsc.VectorSubcoreMesh(core_axis_name="c", subcore_axis_name="s")
    sc_fn = pl.kernel(
        functools.partial(_sc_edge_body, n, d, ept, g_size),
        out_type=[
            jax.ShapeDtypeStruct((NC, n, d), jnp.float32),
            jax.ShapeDtypeStruct((NC, n, 16), jnp.float32),
        ],
        mesh=mesh,
        compiler_params=pltpu.CompilerParams(use_tc_tiling_on_sc=False),
        scratch_types=[
            pltpu.VMEM_SHARED((n, d), jnp.float32),
            pltpu.VMEM_SHARED((n, 16), jnp.float32),
            pltpu.VMEM((n_groups // 2, g_size), jnp.int32),
            pltpu.VMEM((n_groups // 2, g_size), jnp.int32),
        ]
        + [pltpu.VMEM((g_size, 16), jnp.float32)] * 2
        + [pltpu.VMEM((g_size, 16), jnp.float32)] * 2
        + [pltpu.VMEM((g_size, d), jnp.float32)] * 2
        + [pltpu.VMEM((g_size, 16), jnp.float32)] * 2
        + [pltpu.VMEM((g_size, d), jnp.float32)] * 2
        + [
            pltpu.VMEM((16,), jnp.float32),
        ]
        + [pltpu.SemaphoreType.DMA] * 4,
    )
    acc, den = sc_fn(src3, dst3, t1, t2, h_arr, m)

    out = pl.pallas_call(
        _fin_body,
        out_shape=jax.ShapeDtypeStruct((n, d), jnp.float32),
    )(acc, den, sel, bias2)
    return out


# speedup vs baseline: 141.6036x; 2.1529x over previous
import functools

import jax
import jax.numpy as jnp
from jax import lax
from jax.experimental import pallas as pl
from jax.experimental.pallas import tpu as pltpu
from jax.experimental.pallas import tpu_sc as plsc

NC = 2
NS = 16
NW = NC * NS


def _prep_body(x_ref, wf_ref, asrc_ref, adst_ref, h_ref, t1_ref, t2_ref, m_ref):
    h = jnp.dot(x_ref[...], wf_ref[...], preferred_element_type=jnp.float32)
    h_ref[...] = h
    t1 = jnp.dot(h, asrc_ref[...], preferred_element_type=jnp.float32)
    t2 = jnp.dot(h, adst_ref[...], preferred_element_type=jnp.float32)
    t1_ref[...] = t1
    t2_ref[...] = t2
    s = jnp.max(t1, axis=0, keepdims=True) + jnp.max(t2, axis=0, keepdims=True)
    m = jnp.maximum(s, 0.2 * s)
    m_ref[...] = jnp.broadcast_to(m, (8, 16))


def _fin_body(acc_ref, den_ref, sel_ref, bias_ref, out_ref):
    den8 = den_ref[0][:, :8] + den_ref[1][:, :8]
    rep = jnp.dot(den8, sel_ref[...], preferred_element_type=jnp.float32)
    out_ref[...] = (acc_ref[0] + acc_ref[1]) / (rep + 1e-16) + bias_ref[...]


def _sc_edge_body(n_nodes, d_model, ept, g_size,
                  src_hbm, dst_hbm, t1_hbm, t2_hbm, h_hbm, m_hbm,
                  acc_hbm, den_hbm,
                  out_sh, den_sh,
                  sidx_all, didx_all,
                  t1b0, t1b1, t2b0, t2b1,
                  hb0, hb1, zb0, zb1, mg0, mg1,
                  mb,
                  gs0, gs1, ss0, ss1):
    t1bs = (t1b0, t1b1)
    t2bs = (t2b0, t2b1)
    hbs = (hb0, hb1)
    zbs = (zb0, zb1)
    msgbs = (mg0, mg1)
    gsems = (gs0, gs1)
    ssems = (ss0, ss1)
    c = lax.axis_index("c")
    s = lax.axis_index("s")
    wid = c * NS + s
    n_groups = ept // g_size
    n_seg = d_model // 16
    msgb = mg0
    rpt = 8 * (n_nodes // (8 * NS))
    rbase = s * rpt
    tail = n_nodes - NS * rpt
    tail_base = NS * rpt

    zvec = jnp.zeros((16,), jnp.float32)

    def zero_z_row(i, _):
        zb0[i] = zvec
        return 0
    lax.fori_loop(0, g_size, zero_z_row, 0)

    def zero_msg_row(i, _):
        for j in range(n_seg):
            msgb[i, pl.ds(j * 16, 16)] = zvec
        return 0
    lax.fori_loop(0, g_size, zero_msg_row, 0)
    n_zcp = rpt // g_size
    for k in range(n_zcp):
        pltpu.sync_copy(zb0, den_sh.at[pl.ds(rbase + k * g_size, g_size)])
        pltpu.sync_copy(msgb, out_sh.at[pl.ds(rbase + k * g_size, g_size)])
    rem = rpt - n_zcp * g_size
    if rem > 0:
        pltpu.sync_copy(zb0.at[pl.ds(0, rem)],
                        den_sh.at[pl.ds(rbase + n_zcp * g_size, rem)])
        pltpu.sync_copy(msgb.at[pl.ds(0, rem)],
                        out_sh.at[pl.ds(rbase + n_zcp * g_size, rem)])
    if tail > 0:
        @pl.when(s == 0)
        def _zero_tail():
            pltpu.sync_copy(zb0.at[pl.ds(0, tail)],
                            den_sh.at[pl.ds(tail_base, tail)])
            pltpu.sync_copy(msgb.at[pl.ds(0, tail)],
                            out_sh.at[pl.ds(tail_base, tail)])
    plsc.subcore_barrier()

    pltpu.sync_copy(m_hbm.at[0], mb)
    mvec = mb[...]

    def fire_gathers(g, b):
        pltpu.async_copy(t1_hbm.at[sidx_all.at[g]], t1bs[b], gsems[b])
        pltpu.async_copy(t2_hbm.at[didx_all.at[g]], t2bs[b], gsems[b])
        pltpu.async_copy(h_hbm.at[sidx_all.at[g]], hbs[b], gsems[b])

    def wait_gathers(g, b):
        pltpu.make_async_copy(t1_hbm.at[sidx_all.at[g]], t1bs[b], gsems[b]).wait()
        pltpu.make_async_copy(t2_hbm.at[didx_all.at[g]], t2bs[b], gsems[b]).wait()
        pltpu.make_async_copy(h_hbm.at[sidx_all.at[g]], hbs[b], gsems[b]).wait()

    def fire_scatters(g, b):
        pltpu.async_copy(zbs[b], den_sh.at[didx_all.at[g]], ssems[b], add=True)
        pltpu.async_copy(msgbs[b], out_sh.at[didx_all.at[g]], ssems[b], add=True)

    def wait_scatters(g, b):
        pltpu.make_async_copy(zbs[b], den_sh.at[didx_all.at[g]], ssems[b]).wait()
        pltpu.make_async_copy(msgbs[b], out_sh.at[didx_all.at[g]], ssems[b]).wait()

    def compute(g, b):
        t1r, t2r, hr, zr, mr = t1bs[b], t2bs[b], hbs[b], zbs[b], msgbs[b]

        def pair(p, _):
            i0 = 2 * p
            svs = [t1r[i0 + k] + t2r[i0 + k] for k in range(2)]
            hsegs = [[hr[i0 + k, pl.ds(j * 16, 16)] for j in range(n_seg)]
                     for k in range(2)]
            zs = [jnp.exp(jnp.maximum(sv, 0.2 * sv) - mvec) for sv in svs]
            for k in range(2):
                zr[i0 + k] = zs[k]
            for k in range(2):
                for j in range(n_seg):
                    mr[i0 + k, pl.ds(j * 16, 16)] = hsegs[k][j] * zs[k][j]
            return 0
        lax.fori_loop(0, g_size // 2, pair, 0)

    def step(g, b):
        wait_gathers(g, b)

        @pl.when(g >= 2)
        def _():
            wait_scatters(g - 2, b)
        compute(g, b)
        fire_scatters(g, b)

    n_phases = 2
    assert n_groups % n_phases == 0
    pg = n_groups // n_phases
    assert pg > 2
    for ph in range(n_phases):
        pltpu.sync_copy(src_hbm.at[wid, pl.ds(ph * pg, pg)], sidx_all)
        pltpu.sync_copy(dst_hbm.at[wid, pl.ds(ph * pg, pg)], didx_all)
        for b in range(2):
            fire_gathers(b, b)

        def body(i, _):
            for b in range(2):
                g = 2 * i + b
                step(g, b)

                @pl.when(g < pg - 2)
                def _():
                    fire_gathers(g + 2, b)
            return 0
        lax.fori_loop(0, pg // 2, body, 0)
        if pg % 2 == 1:
            step(pg - 1, (pg - 1) % 2)
        for g in (pg - 2, pg - 1):
            wait_scatters(g, g % 2)

    plsc.subcore_barrier()
    pltpu.sync_copy(out_sh.at[pl.ds(rbase, rpt)],
                    acc_hbm.at[c, pl.ds(rbase, rpt)])
    pltpu.sync_copy(den_sh.at[pl.ds(rbase, rpt)],
                    den_hbm.at[c, pl.ds(rbase, rpt)])
    if tail > 0:
        @pl.when(s == 0)
        def _read_tail():
            pltpu.sync_copy(out_sh.at[pl.ds(tail_base, tail)],
                            acc_hbm.at[c, pl.ds(tail_base, tail)])
            pltpu.sync_copy(den_sh.at[pl.ds(tail_base, tail)],
                            den_hbm.at[c, pl.ds(tail_base, tail)])


def kernel(x, edge_index, W, a_src, a_dst, bias):
    n, f_in = x.shape
    h_heads, _, f_out = W.shape
    e_edges = edge_index.shape[1]
    d = h_heads * f_out

    wf = W.transpose(1, 0, 2).reshape(f_in, d)
    rows = jnp.arange(d)
    hcol = rows // f_out
    a_src_m = jnp.zeros((d, 16), jnp.float32).at[rows, hcol].set(a_src.reshape(-1))
    a_dst_m = jnp.zeros((d, 16), jnp.float32).at[rows, hcol].set(a_dst.reshape(-1))
    src = edge_index[0].astype(jnp.int32)
    dst = edge_index[1].astype(jnp.int32)
    sel = jnp.repeat(jnp.eye(h_heads, dtype=jnp.float32), f_out, axis=1)
    bias2 = bias.reshape(1, d).astype(jnp.float32)

    h_arr, t1, t2, m = pl.pallas_call(
        _prep_body,
        out_shape=[
            jax.ShapeDtypeStruct((n, d), jnp.float32),
            jax.ShapeDtypeStruct((n, 16), jnp.float32),
            jax.ShapeDtypeStruct((n, 16), jnp.float32),
            jax.ShapeDtypeStruct((8, 16), jnp.float32),
        ],
    )(x, wf, a_src_m, a_dst_m)

    ept = e_edges // NW
    g_size = 40
    assert ept % g_size == 0 and n % NS == 0

    n_groups = ept // g_size
    src3 = src.reshape(NW, n_groups, g_size)
    dst3 = dst.reshape(NW, n_groups, g_size)

    mesh = plsc.VectorSubcoreMesh(core_axis_name="c", subcore_axis_name="s")
    sc_fn = pl.kernel(
        functools.partial(_sc_edge_body, n, d, ept, g_size),
        out_type=[
            jax.ShapeDtypeStruct((NC, n, d), jnp.float32),
            jax.ShapeDtypeStruct((NC, n, 16), jnp.float32),
        ],
        mesh=mesh,
        compiler_params=pltpu.CompilerParams(use_tc_tiling_on_sc=False),
        scratch_types=[
            pltpu.VMEM_SHARED((n, d), jnp.float32),
            pltpu.VMEM_SHARED((n, 16), jnp.float32),
            pltpu.VMEM((n_groups // 2, g_size), jnp.int32),
            pltpu.VMEM((n_groups // 2, g_size), jnp.int32),
        ]
        + [pltpu.VMEM((g_size, 16), jnp.float32)] * 2
        + [pltpu.VMEM((g_size, 16), jnp.float32)] * 2
        + [pltpu.VMEM((g_size, d), jnp.float32)] * 2
        + [pltpu.VMEM((g_size, 16), jnp.float32)] * 2
        + [pltpu.VMEM((g_size, d), jnp.float32)] * 2
        + [
            pltpu.VMEM((16,), jnp.float32),
        ]
        + [pltpu.SemaphoreType.DMA] * 4,
    )
    acc, den = sc_fn(src3, dst3, t1, t2, h_arr, m)

    out = pl.pallas_call(
        _fin_body,
        out_shape=jax.ShapeDtypeStruct((n, d), jnp.float32),
    )(acc, den, sel, bias2)
    return out
